# baseline (speedup 1.0000x reference)
import functools

import jax
import jax.numpy as jnp
from jax import lax
from jax.experimental import pallas as pl
from jax.experimental.pallas import tpu as pltpu
from jax.experimental.pallas import tpu_sc as plsc

F = 128
NC = 2
NS = 16
NW = NC * NS


def _rows(n, pref):
    return pref if n % pref == 0 else n


def _proj_body(env_ref, w_ref, b_ref, t_ref):
    t_ref[...] = (
        jnp.dot(env_ref[...], w_ref[...], preferred_element_type=jnp.float32)
        + b_ref[0])


def _proj(env, wall, bias2):
    n = env.shape[0]
    r = _rows(n, 1000)
    nb = n // r
    return pl.pallas_call(
        _proj_body,
        grid=(2, nb),
        in_specs=[
            pl.BlockSpec((r, F), lambda j, i: (i, 0)),
            pl.BlockSpec((F, 2 * F), lambda j, i: (0, j)),
            pl.BlockSpec((1, 1, 2 * F), lambda j, i: (j, 0, 0)),
        ],
        out_specs=pl.BlockSpec((r, 2 * F), lambda j, i: (j * nb + i, 0)),
        out_shape=jax.ShapeDtypeStruct((2 * n, 2 * F), jnp.float32),
    )(env, wall, bias2)


def _ea_body(ea_ref, w_ref, o1, o2, o3):
    p = jnp.dot(ea_ref[...], w_ref[...], preferred_element_type=jnp.float32)
    o1[...] = p[:, : 2 * F]
    o2[...] = p[:, 2 * F : 4 * F]
    o3[...] = p[:, 4 * F :]


def _ea_proj(ea, w3):
    e, k = ea.shape
    r = _rows(e, 2000)
    return pl.pallas_call(
        _ea_body,
        grid=(e // r,),
        in_specs=[
            pl.BlockSpec((r, k), lambda i: (i, 0)),
            pl.BlockSpec((k, 6 * F), lambda i: (0, 0)),
        ],
        out_specs=[pl.BlockSpec((r, 2 * F), lambda i: (i, 0))] * 3,
        out_shape=[jax.ShapeDtypeStruct((e, 2 * F), jnp.float32)] * 3,
    )(ea, w3)


def _self_msg(s, d):
    gf = s[:, :F] + d[:, :F]
    gs = s[:, F:] + d[:, F:]
    sig = 1.0 / (1.0 + jnp.exp(-gf))
    sp = jnp.maximum(gs, 0.0) + jnp.log(1.0 + jnp.exp(-jnp.abs(gs)))
    return sig * sp


def _update_body(e_ref, p_ref, s_ref, d_ref, o_ref):
    o_ref[...] = (
        e_ref[...]
        + p_ref[0]
        + p_ref[1]
        + _self_msg(s_ref[...], d_ref[...])
    )


def _update(env, part, t):
    n = env.shape[0]
    r = _rows(n, 1000)
    nb = n // r
    return pl.pallas_call(
        _update_body,
        grid=(nb,),
        in_specs=[
            pl.BlockSpec((r, F), lambda i: (i, 0)),
            pl.BlockSpec((NC, r, F), lambda i: (0, i, 0)),
            pl.BlockSpec((r, 2 * F), lambda i: (i, 0)),
            pl.BlockSpec((r, 2 * F), lambda i: (nb + i, 0)),
        ],
        out_specs=pl.BlockSpec((r, F), lambda i: (i, 0)),
        out_shape=jax.ShapeDtypeStruct((n, F), jnp.float32),
    )(env, part, t, t)


def _final_body(e_ref, p_ref, s_ref, d_ref, w21_ref, b21_ref, w22_ref,
                b22_ref, w23_ref, b23_ref, o_ref, acc_ref, *, n):
    i = pl.program_id(0)

    @pl.when(i == 0)
    def _():
        acc_ref[...] = jnp.zeros_like(acc_ref)

    env4 = (
        e_ref[...]
        + p_ref[0]
        + p_ref[1]
        + _self_msg(s_ref[...], d_ref[...])
    )
    acc_ref[...] += jnp.sum(env4, axis=0, keepdims=True)

    @pl.when(i == pl.num_programs(0) - 1)
    def _():
        pooled = acc_ref[...] * (1.0 / n)
        h1 = jnp.maximum(
            jnp.dot(pooled, w21_ref[...], preferred_element_type=jnp.float32)
            + b21_ref[...], 0.0)
        h2 = jnp.maximum(
            jnp.dot(h1, w22_ref[...], preferred_element_type=jnp.float32)
            + b22_ref[...], 0.0)
        o_ref[...] = (
            jnp.sum(h2 * w23_ref[...], axis=1, keepdims=True) + b23_ref[...]
        )


def _final(env, part, t, w21, b21, w22, b22, w23r, b23r):
    n = env.shape[0]
    r = _rows(n, 1000)
    nb = n // r
    return pl.pallas_call(
        functools.partial(_final_body, n=n),
        grid=(nb,),
        in_specs=[
            pl.BlockSpec((r, F), lambda i: (i, 0)),
            pl.BlockSpec((NC, r, F), lambda i: (0, i, 0)),
            pl.BlockSpec((r, 2 * F), lambda i: (i, 0)),
            pl.BlockSpec((r, 2 * F), lambda i: (nb + i, 0)),
            pl.BlockSpec((F, 2 * F), lambda i: (0, 0)),
            pl.BlockSpec((1, 2 * F), lambda i: (0, 0)),
            pl.BlockSpec((2 * F, F), lambda i: (0, 0)),
            pl.BlockSpec((1, F), lambda i: (0, 0)),
            pl.BlockSpec((1, F), lambda i: (0, 0)),
            pl.BlockSpec((1, 1), lambda i: (0, 0)),
        ],
        out_specs=pl.BlockSpec((1, 1), lambda i: (0, 0)),
        out_shape=jax.ShapeDtypeStruct((1, 1), jnp.float32),
        scratch_shapes=[pltpu.VMEM((1, F), jnp.float32)],
        compiler_params=pltpu.CompilerParams(
            dimension_semantics=("arbitrary",)),
    )(env, part, t, t, w21, b21, w22, b22, w23r, b23r)


_B = 16


def _gate16(bt, be, bm, rr):
    c8, c7, c6, c5, c4 = (0.0051261021414032125, -0.02907406467853027,
                          0.07751608674076167, -0.13602247622393474,
                          0.19076880735651539)
    c3, c2, c1, c0 = (-0.24835398988480129, 0.3331812170752912,
                      -0.49999444976340335, 0.9999999659255092)
    for cc in range(F // 16):
        o = cc * 16
        gf = (bt[rr, pl.ds(o, 16)] + bt[_B + rr, pl.ds(o, 16)]
              + be[rr, pl.ds(o, 16)])
        gs = (bt[rr, pl.ds(F + o, 16)] + bt[_B + rr, pl.ds(F + o, 16)]
              + be[rr, pl.ds(F + o, 16)])
        ef = jnp.exp(-gf)
        t = jnp.exp(jnp.minimum(gs, -gs))
        p = c8
        for cx in (c7, c6, c5, c4, c3, c2, c1, c0):
            p = p * t + cx
        sp = jnp.maximum(gs, 0.0) + t * p
        bm[rr, pl.ds(o, 16)] = sp / (1.0 + ef)


def _sc_body(t_hbm, ea_hbm, pk_hbm, z_hbm, out_hbm,
             pk_all, bt0, be0, bm0, bt1, be1, bm1,
             acc, st0, se0, sm0, st1, se1, sm1, *, n, e):
    c = lax.axis_index("c")
    s = lax.axis_index("s")
    wid = s * NC + c
    epw = e // NW
    chunks = epw // _B
    rpt = (n // NS) // 8 * 8
    rem = n - NS * rpt
    r0 = pl.multiple_of(s * rpt, 8)
    pltpu.sync_copy(z_hbm.at[pl.ds(r0, rpt)], acc.at[pl.ds(r0, rpt)])
    if rem:
        @pl.when(s == 0)
        def _():
            pltpu.sync_copy(z_hbm.at[pl.ds(NS * rpt, rem)],
                            acc.at[pl.ds(NS * rpt, rem)])
    plsc.subcore_barrier()

    base = pl.multiple_of(wid * epw, 8)
    pltpu.sync_copy(pk_hbm.at[pl.ds(2 * base, 2 * epw)], pk_all)

    parity = ((bt0, be0, bm0, st0, se0, sm0),
              (bt1, be1, bm1, st1, se1, sm1))

    def issue(g, p):
        bt, be, _, s_t, s_e, _ = parity[p]
        io2 = pl.multiple_of(g * 2 * _B, 8)
        pltpu.async_copy(t_hbm.at[pk_all.at[pl.ds(io2, 2 * _B)]], bt, s_t)
        off = pl.multiple_of(base + g * _B, 8)
        pltpu.async_copy(ea_hbm.at[pl.ds(off, _B)], be, s_e)

    def consume(g, p):
        bt, be, bm, s_t, s_e, s_m = parity[p]
        io2 = pl.multiple_of(g * 2 * _B, 8)
        pltpu.make_async_copy(
            t_hbm.at[pk_all.at[pl.ds(io2, 2 * _B)]], bt, s_t).wait()
        pltpu.make_async_copy(ea_hbm.at[pl.ds(0, _B)], be, s_e).wait()
        dstv = pk_all[pl.ds(io2 + _B, _B)] - n

        @pl.when(g >= 2)
        def _():
            pltpu.make_async_copy(bm, acc.at[dstv], s_m).wait()

        def _row(rr, carry2):
            _gate16(bt, be, bm, rr)
            return carry2

        lax.fori_loop(0, _B, _row, 0)

        pltpu.async_copy(bm, acc.at[dstv], s_m, add=True)

    issue(0, 0)

    def pair(k, carry):
        issue(2 * k + 1, 1)
        consume(2 * k, 0)
        issue(2 * k + 2, 0)
        consume(2 * k + 1, 1)
        return carry

    lax.fori_loop(0, (chunks - 1) // 2, pair, 0)
    consume(chunks - 1, 0)
    dv = pk_all[pl.ds(0, _B)] - n
    pltpu.make_async_copy(bm1, acc.at[dv], sm1).wait()
    pltpu.make_async_copy(bm0, acc.at[dv], sm0).wait()

    plsc.subcore_barrier()
    pltpu.sync_copy(acc.at[pl.ds(r0, rpt)], out_hbm.at[c, pl.ds(r0, rpt)])
    if rem:
        @pl.when(s == 0)
        def _():
            pltpu.sync_copy(acc.at[pl.ds(NS * rpt, rem)],
                            out_hbm.at[c, pl.ds(NS * rpt, rem)])


def _sc_edge(t, ea, pk, zeros):
    n = t.shape[0] // 2
    e = pk.shape[0] // 2
    epw = e // NW
    mesh = plsc.VectorSubcoreMesh(
        core_axis_name="c", subcore_axis_name="s",
        num_cores=NC, num_subcores=NS)
    kern = pl.kernel(
        functools.partial(_sc_body, n=n, e=e),
        out_type=jax.ShapeDtypeStruct((NC, n, F), jnp.float32),
        mesh=mesh,
        scratch_types=[
            pltpu.VMEM((2 * epw,), jnp.int32),
            pltpu.VMEM((2 * _B, 2 * F), jnp.float32),
            pltpu.VMEM((_B, 2 * F), jnp.float32),
            pltpu.VMEM((_B, F), jnp.float32),
            pltpu.VMEM((2 * _B, 2 * F), jnp.float32),
            pltpu.VMEM((_B, 2 * F), jnp.float32),
            pltpu.VMEM((_B, F), jnp.float32),
            pltpu.VMEM_SHARED((n, F), jnp.float32),
            pltpu.SemaphoreType.DMA,
            pltpu.SemaphoreType.DMA,
            pltpu.SemaphoreType.DMA,
            pltpu.SemaphoreType.DMA,
            pltpu.SemaphoreType.DMA,
            pltpu.SemaphoreType.DMA,
        ],
    )
    return kern(t, ea, pk, zeros)


def kernel(x, edge_index, edge_attr, wf1, bf1, ws1, bs1, wf2, bf2, ws2, bs2,
           wf3, bf3, ws3, bs3, w21, b21, w22, b22, w23, b23):
    n = x.shape[0]
    src = edge_index[0]
    dst = edge_index[1]
    zeros = jnp.zeros((n, F), jnp.float32)

    def mk(wf, ws, bf, bs):
        wall = jnp.concatenate(
            [wf[:F], ws[:F], wf[F:2 * F], ws[F:2 * F]], axis=1)
        brow = jnp.concatenate([bf, bs]).reshape(1, 1, 2 * F)
        bias2 = jnp.concatenate([jnp.zeros_like(brow), brow], axis=0)
        wea = jnp.concatenate([wf[2 * F:], ws[2 * F:]], axis=1)
        return wall, bias2, wea

    wall1, bias1, wea1 = mk(wf1, ws1, bf1, bs1)
    wall2, bias2, wea2 = mk(wf2, ws2, bf2, bs2)
    wall3, bias3, wea3 = mk(wf3, ws3, bf3, bs3)

    pk = jnp.concatenate(
        [src.reshape(-1, _B), dst.reshape(-1, _B) + n], axis=1).reshape(-1)

    ea1, ea2, ea3 = _ea_proj(
        edge_attr, jnp.concatenate([wea1, wea2, wea3], axis=1))

    t1 = _proj(x, wall1, bias1)
    p1 = _sc_edge(t1, ea1, pk, zeros)
    env2 = _update(x, p1, t1)

    t2 = _proj(env2, wall2, bias2)
    p2 = _sc_edge(t2, ea2, pk, zeros)
    env3 = _update(env2, p2, t2)

    t3 = _proj(env3, wall3, bias3)
    p3 = _sc_edge(t3, ea3, pk, zeros)

    return _final(env3, p3, t3,
                  w21, b21.reshape(1, 2 * F),
                  w22, b22.reshape(1, F),
                  w23.reshape(1, F), b23.reshape(1, 1))


# speedup vs baseline: 1.3851x; 1.3851x over previous
import functools

import jax
import jax.numpy as jnp
from jax import lax
from jax.experimental import pallas as pl
from jax.experimental.pallas import tpu as pltpu
from jax.experimental.pallas import tpu_sc as plsc

F = 128
NC = 2
NS = 16
NW = NC * NS


def _rows(n, pref):
    return pref if n % pref == 0 else n


def _proj_body(env_ref, w_ref, b_ref, t_ref):
    t_ref[...] = (
        jnp.dot(env_ref[...], w_ref[...], preferred_element_type=jnp.float32)
        + b_ref[0])


def _proj(env, wall, bias2):
    n = env.shape[0]
    r = _rows(n, 1000)
    nb = n // r
    return pl.pallas_call(
        _proj_body,
        grid=(2, nb),
        in_specs=[
            pl.BlockSpec((r, F), lambda j, i: (i, 0)),
            pl.BlockSpec((F, 2 * F), lambda j, i: (0, j)),
            pl.BlockSpec((1, 1, 2 * F), lambda j, i: (j, 0, 0)),
        ],
        out_specs=pl.BlockSpec((r, 2 * F), lambda j, i: (j * nb + i, 0)),
        out_shape=jax.ShapeDtypeStruct((2 * n, 2 * F), jnp.float32),
    )(env, wall, bias2)


def _ea_body(ea_ref, w_ref, o1, o2, o3):
    p = jnp.dot(ea_ref[...], w_ref[...], preferred_element_type=jnp.float32)
    o1[...] = p[:, : 2 * F]
    o2[...] = p[:, 2 * F : 4 * F]
    o3[...] = p[:, 4 * F :]


def _ea_proj(ea, w3):
    e, k = ea.shape
    r = _rows(e, 2000)
    return pl.pallas_call(
        _ea_body,
        grid=(e // r,),
        in_specs=[
            pl.BlockSpec((r, k), lambda i: (i, 0)),
            pl.BlockSpec((k, 6 * F), lambda i: (0, 0)),
        ],
        out_specs=[pl.BlockSpec((r, 2 * F), lambda i: (i, 0))] * 3,
        out_shape=[jax.ShapeDtypeStruct((e, 2 * F), jnp.float32)] * 3,
    )(ea, w3)


def _self_msg(s, d):
    gf = s[:, :F] + d[:, :F]
    gs = s[:, F:] + d[:, F:]
    sig = 1.0 / (1.0 + jnp.exp(-gf))
    sp = jnp.maximum(gs, 0.0) + jnp.log(1.0 + jnp.exp(-jnp.abs(gs)))
    return sig * sp


def _update_body(e_ref, p_ref, s_ref, d_ref, o_ref):
    o_ref[...] = (
        e_ref[...]
        + p_ref[0]
        + p_ref[1]
        + _self_msg(s_ref[...], d_ref[...])
    )


def _update(env, part, t):
    n = env.shape[0]
    r = _rows(n, 1000)
    nb = n // r
    return pl.pallas_call(
        _update_body,
        grid=(nb,),
        in_specs=[
            pl.BlockSpec((r, F), lambda i: (i, 0)),
            pl.BlockSpec((NC, r, F), lambda i: (0, i, 0)),
            pl.BlockSpec((r, 2 * F), lambda i: (i, 0)),
            pl.BlockSpec((r, 2 * F), lambda i: (nb + i, 0)),
        ],
        out_specs=pl.BlockSpec((r, F), lambda i: (i, 0)),
        out_shape=jax.ShapeDtypeStruct((n, F), jnp.float32),
    )(env, part, t, t)


def _final_body(e_ref, p_ref, s_ref, d_ref, w21_ref, b21_ref, w22_ref,
                b22_ref, w23_ref, b23_ref, o_ref, acc_ref, *, n):
    i = pl.program_id(0)

    @pl.when(i == 0)
    def _():
        acc_ref[...] = jnp.zeros_like(acc_ref)

    env4 = (
        e_ref[...]
        + p_ref[0]
        + p_ref[1]
        + _self_msg(s_ref[...], d_ref[...])
    )
    acc_ref[...] += jnp.sum(env4, axis=0, keepdims=True)

    @pl.when(i == pl.num_programs(0) - 1)
    def _():
        pooled = acc_ref[...] * (1.0 / n)
        h1 = jnp.maximum(
            jnp.dot(pooled, w21_ref[...], preferred_element_type=jnp.float32)
            + b21_ref[...], 0.0)
        h2 = jnp.maximum(
            jnp.dot(h1, w22_ref[...], preferred_element_type=jnp.float32)
            + b22_ref[...], 0.0)
        o_ref[...] = (
            jnp.sum(h2 * w23_ref[...], axis=1, keepdims=True) + b23_ref[...]
        )


def _final(env, part, t, w21, b21, w22, b22, w23r, b23r):
    n = env.shape[0]
    r = _rows(n, 1000)
    nb = n // r
    return pl.pallas_call(
        functools.partial(_final_body, n=n),
        grid=(nb,),
        in_specs=[
            pl.BlockSpec((r, F), lambda i: (i, 0)),
            pl.BlockSpec((NC, r, F), lambda i: (0, i, 0)),
            pl.BlockSpec((r, 2 * F), lambda i: (i, 0)),
            pl.BlockSpec((r, 2 * F), lambda i: (nb + i, 0)),
            pl.BlockSpec((F, 2 * F), lambda i: (0, 0)),
            pl.BlockSpec((1, 2 * F), lambda i: (0, 0)),
            pl.BlockSpec((2 * F, F), lambda i: (0, 0)),
            pl.BlockSpec((1, F), lambda i: (0, 0)),
            pl.BlockSpec((1, F), lambda i: (0, 0)),
            pl.BlockSpec((1, 1), lambda i: (0, 0)),
        ],
        out_specs=pl.BlockSpec((1, 1), lambda i: (0, 0)),
        out_shape=jax.ShapeDtypeStruct((1, 1), jnp.float32),
        scratch_shapes=[pltpu.VMEM((1, F), jnp.float32)],
        compiler_params=pltpu.CompilerParams(
            dimension_semantics=("arbitrary",)),
    )(env, part, t, t, w21, b21, w22, b22, w23r, b23r)


_B = 16


def _gate16(bt, be, bm, rr):
    l0, l1, l2, l3, l4, l5 = (0.9999905920354432, -0.499314652271837,
                              0.32484958072281855, -0.20907953599756066,
                              0.10013973265606613, -0.02344365580418084)
    r0_, r1_, r2_, r3_, r4_, r5_ = (0.9999489821947859, -0.9962757002193762,
                                    0.9534207438395886, -0.7705729299525199,
                                    0.4179378603132159, -0.10448446507830325)
    for cc in range(F // 16):
        o = cc * 16
        gf = (bt[rr, pl.ds(o, 16)] + bt[_B + rr, pl.ds(o, 16)]
              + be[rr, pl.ds(o, 16)])
        gs = (bt[rr, pl.ds(F + o, 16)] + bt[_B + rr, pl.ds(F + o, 16)]
              + be[rr, pl.ds(F + o, 16)])
        u = jnp.exp(jnp.minimum(gf, -gf))
        u2 = u * u
        q = (r0_ + r1_ * u) + u2 * ((r2_ + r3_ * u) + u2 * (r4_ + r5_ * u))
        sig = jnp.where(gf >= 0.0, q, 1.0 - q)
        t = jnp.exp(jnp.minimum(gs, -gs))
        t2 = t * t
        pl1 = (l0 + l1 * t) + t2 * ((l2 + l3 * t) + t2 * (l4 + l5 * t))
        sp = jnp.maximum(gs, 0.0) + t * pl1
        bm[rr, pl.ds(o, 16)] = sig * sp


def _sc_body(t_hbm, ea_hbm, pk_hbm, z_hbm, out_hbm,
             pk_all, bt0, be0, bm0, bt1, be1, bm1,
             acc, st0, se0, sm0, st1, se1, sm1, *, n, e):
    c = lax.axis_index("c")
    s = lax.axis_index("s")
    wid = s * NC + c
    epw = e // NW
    chunks = epw // _B
    rpt = (n // NS) // 8 * 8
    rem = n - NS * rpt
    r0 = pl.multiple_of(s * rpt, 8)
    pltpu.sync_copy(z_hbm.at[pl.ds(r0, rpt)], acc.at[pl.ds(r0, rpt)])
    if rem:
        @pl.when(s == 0)
        def _():
            pltpu.sync_copy(z_hbm.at[pl.ds(NS * rpt, rem)],
                            acc.at[pl.ds(NS * rpt, rem)])
    plsc.subcore_barrier()

    base = pl.multiple_of(wid * epw, 8)
    pltpu.sync_copy(pk_hbm.at[pl.ds(2 * base, 2 * epw)], pk_all)

    parity = ((bt0, be0, bm0, st0, se0, sm0),
              (bt1, be1, bm1, st1, se1, sm1))

    def issue(g, p):
        bt, be, _, s_t, s_e, _ = parity[p]
        io2 = pl.multiple_of(g * 2 * _B, 8)
        pltpu.async_copy(t_hbm.at[pk_all.at[pl.ds(io2, 2 * _B)]], bt, s_t)
        off = pl.multiple_of(base + g * _B, 8)
        pltpu.async_copy(ea_hbm.at[pl.ds(off, _B)], be, s_e)

    def consume(g, p):
        bt, be, bm, s_t, s_e, s_m = parity[p]
        io2 = pl.multiple_of(g * 2 * _B, 8)
        pltpu.make_async_copy(
            t_hbm.at[pk_all.at[pl.ds(io2, 2 * _B)]], bt, s_t).wait()
        pltpu.make_async_copy(ea_hbm.at[pl.ds(0, _B)], be, s_e).wait()
        dstv = pk_all[pl.ds(io2 + _B, _B)] - n

        @pl.when(g >= 2)
        def _():
            pltpu.make_async_copy(bm, acc.at[dstv], s_m).wait()

        def _row(rr, carry2):
            _gate16(bt, be, bm, rr)
            return carry2

        lax.fori_loop(0, _B, _row, 0)

        pltpu.async_copy(bm, acc.at[dstv], s_m, add=True)

    issue(0, 0)

    def pair(k, carry):
        issue(2 * k + 1, 1)
        consume(2 * k, 0)
        issue(2 * k + 2, 0)
        consume(2 * k + 1, 1)
        return carry

    lax.fori_loop(0, (chunks - 1) // 2, pair, 0)
    consume(chunks - 1, 0)
    dv = pk_all[pl.ds(0, _B)] - n
    pltpu.make_async_copy(bm1, acc.at[dv], sm1).wait()
    pltpu.make_async_copy(bm0, acc.at[dv], sm0).wait()

    plsc.subcore_barrier()
    pltpu.sync_copy(acc.at[pl.ds(r0, rpt)], out_hbm.at[c, pl.ds(r0, rpt)])
    if rem:
        @pl.when(s == 0)
        def _():
            pltpu.sync_copy(acc.at[pl.ds(NS * rpt, rem)],
                            out_hbm.at[c, pl.ds(NS * rpt, rem)])


def _sc_edge(t, ea, pk, zeros):
    n = t.shape[0] // 2
    e = pk.shape[0] // 2
    epw = e // NW
    mesh = plsc.VectorSubcoreMesh(
        core_axis_name="c", subcore_axis_name="s",
        num_cores=NC, num_subcores=NS)
    kern = pl.kernel(
        functools.partial(_sc_body, n=n, e=e),
        out_type=jax.ShapeDtypeStruct((NC, n, F), jnp.float32),
        mesh=mesh,
        scratch_types=[
            pltpu.VMEM((2 * epw,), jnp.int32),
            pltpu.VMEM((2 * _B, 2 * F), jnp.float32),
            pltpu.VMEM((_B, 2 * F), jnp.float32),
            pltpu.VMEM((_B, F), jnp.float32),
            pltpu.VMEM((2 * _B, 2 * F), jnp.float32),
            pltpu.VMEM((_B, 2 * F), jnp.float32),
            pltpu.VMEM((_B, F), jnp.float32),
            pltpu.VMEM_SHARED((n, F), jnp.float32),
            pltpu.SemaphoreType.DMA,
            pltpu.SemaphoreType.DMA,
            pltpu.SemaphoreType.DMA,
            pltpu.SemaphoreType.DMA,
            pltpu.SemaphoreType.DMA,
            pltpu.SemaphoreType.DMA,
        ],
    )
    return kern(t, ea, pk, zeros)


def kernel(x, edge_index, edge_attr, wf1, bf1, ws1, bs1, wf2, bf2, ws2, bs2,
           wf3, bf3, ws3, bs3, w21, b21, w22, b22, w23, b23):
    n = x.shape[0]
    src = edge_index[0]
    dst = edge_index[1]
    zeros = jnp.zeros((n, F), jnp.float32)

    def mk(wf, ws, bf, bs):
        wall = jnp.concatenate(
            [wf[:F], ws[:F], wf[F:2 * F], ws[F:2 * F]], axis=1)
        brow = jnp.concatenate([bf, bs]).reshape(1, 1, 2 * F)
        bias2 = jnp.concatenate([jnp.zeros_like(brow), brow], axis=0)
        wea = jnp.concatenate([wf[2 * F:], ws[2 * F:]], axis=1)
        return wall, bias2, wea

    wall1, bias1, wea1 = mk(wf1, ws1, bf1, bs1)
    wall2, bias2, wea2 = mk(wf2, ws2, bf2, bs2)
    wall3, bias3, wea3 = mk(wf3, ws3, bf3, bs3)

    pk = jnp.concatenate(
        [src.reshape(-1, _B), dst.reshape(-1, _B) + n], axis=1).reshape(-1)

    ea1, ea2, ea3 = _ea_proj(
        edge_attr, jnp.concatenate([wea1, wea2, wea3], axis=1))

    t1 = _proj(x, wall1, bias1)
    p1 = _sc_edge(t1, ea1, pk, zeros)
    env2 = _update(x, p1, t1)

    t2 = _proj(env2, wall2, bias2)
    p2 = _sc_edge(t2, ea2, pk, zeros)
    env3 = _update(env2, p2, t2)

    t3 = _proj(env3, wall3, bias3)
    p3 = _sc_edge(t3, ea3, pk, zeros)

    return _final(env3, p3, t3,
                  w21, b21.reshape(1, 2 * F),
                  w22, b22.reshape(1, F),
                  w23.reshape(1, F), b23.reshape(1, 1))
